# baseline (speedup 1.0000x reference)
import functools

import jax
import jax.numpy as jnp
from jax import lax
from jax.experimental import pallas as pl
from jax.experimental.pallas import tpu as pltpu
from jax.experimental.pallas import tpu_sc as plsc

BATCH = 16384
LATENT = 32
HIDDEN = 128
IDX_CHUNK = 128


def _build_gather():
    info = plsc.get_sparse_core_info()
    nc, ns = info.num_cores, info.num_subcores
    nw = nc * ns
    b_per_w = BATCH // nw
    n_chunks = b_per_w // IDX_CHUNK
    mesh = plsc.VectorSubcoreMesh(core_axis_name="c", subcore_axis_name="s")

    @functools.partial(
        pl.kernel,
        mesh=mesh,
        compiler_params=pltpu.CompilerParams(use_tc_tiling_on_sc=False),
        out_type=[
            jax.ShapeDtypeStruct((nw, b_per_w, LATENT), jnp.float32),
            jax.ShapeDtypeStruct((nw, b_per_w, LATENT), jnp.float32),
        ],
        scratch_types=[
            pltpu.VMEM((n_chunks, IDX_CHUNK), jnp.int32),
            pltpu.VMEM((n_chunks, IDX_CHUNK), jnp.int32),
            pltpu.VMEM((b_per_w, LATENT), jnp.float32),
            pltpu.VMEM((b_per_w, LATENT), jnp.float32),
            pltpu.SemaphoreType.DMA,
            pltpu.SemaphoreType.DMA,
        ],
    )
    def gather(user_hbm, item_hbm, utab_hbm, itab_hbm, ue_out, ie_out,
               uidx_v, iidx_v, urows_v, irows_v, usem, isem):
        wid = lax.axis_index("s") * nc + lax.axis_index("c")
        pltpu.sync_copy(user_hbm.at[wid], uidx_v)
        pltpu.sync_copy(item_hbm.at[wid], iidx_v)
        ucopies = []
        icopies = []
        for j in range(n_chunks):
            ucopies.append(pltpu.async_copy(
                utab_hbm.at[uidx_v.at[j]],
                urows_v.at[pl.ds(j * IDX_CHUNK, IDX_CHUNK)], usem))
            icopies.append(pltpu.async_copy(
                itab_hbm.at[iidx_v.at[j]],
                irows_v.at[pl.ds(j * IDX_CHUNK, IDX_CHUNK)], isem))
        for c in ucopies:
            c.wait()
        for c in icopies:
            c.wait()
        pltpu.sync_copy(urows_v, ue_out.at[wid])
        pltpu.sync_copy(irows_v, ie_out.at[wid])

    return gather, nw, b_per_w


_GATHER, _NW, _B_PER_W = _build_gather()


def _mlp_body(ue_ref, ie_ref, w1a_ref, w1b_ref, b1_ref, gamma_ref, beta_ref,
              w2_ref, b2_ref, out_ref):
    h = (jnp.dot(ue_ref[...], w1a_ref[...], preferred_element_type=jnp.float32)
         + jnp.dot(ie_ref[...], w1b_ref[...], preferred_element_type=jnp.float32)
         + b1_ref[...])
    mean = jnp.mean(h, axis=0, keepdims=True)
    var = jnp.mean((h - mean) ** 2, axis=0, keepdims=True)
    hn = (h - mean) * lax.rsqrt(var + 1e-5) * gamma_ref[...] + beta_ref[...]
    hr = jnp.maximum(hn, 0.0)
    logit = jnp.dot(hr, w2_ref[...], preferred_element_type=jnp.float32) + b2_ref[...]
    out_ref[...] = jax.nn.sigmoid(logit)


_MLP = pl.pallas_call(
    _mlp_body,
    out_shape=jax.ShapeDtypeStruct((BATCH, 1), jnp.float32),
)


def kernel(user, item, user_table, item_table, W1, b1, gamma, beta, W2, b2):
    user3 = user.astype(jnp.int32).reshape(_NW, _B_PER_W // IDX_CHUNK, IDX_CHUNK)
    item3 = item.astype(jnp.int32).reshape(_NW, _B_PER_W // IDX_CHUNK, IDX_CHUNK)
    ue, ie = _GATHER(user3, item3, user_table, item_table)
    out = _MLP(
        ue.reshape(BATCH, LATENT), ie.reshape(BATCH, LATENT),
        W1[:LATENT], W1[LATENT:],
        b1.reshape(1, HIDDEN), gamma.reshape(1, HIDDEN), beta.reshape(1, HIDDEN),
        W2, b2.reshape(1, 1),
    )
    return out.reshape(BATCH)


# speedup vs baseline: 1.5866x; 1.5866x over previous
import functools

import jax
import jax.numpy as jnp
from jax import lax
from jax.experimental import pallas as pl
from jax.experimental.pallas import tpu as pltpu
from jax.experimental.pallas import tpu_sc as plsc

BATCH = 16384
LATENT = 32
HIDDEN = 128
NUM_ROWS = 1000000
VP = 262144
SBITS = 18
RBLK = 2048
NBLK = VP // RBLK


def _repack_body(u0, u1, u2, u3, i0, i1, i2, i3, uout, iout):
    uout[...] = jnp.concatenate(
        [jnp.transpose(r[...], (1, 0)) for r in (u0, u1, u2, u3)], axis=1)
    iout[...] = jnp.concatenate(
        [jnp.transpose(r[...], (1, 0)) for r in (i0, i1, i2, i3)], axis=1)


_REPACK = pl.pallas_call(
    _repack_body,
    grid=(NBLK,),
    in_specs=[pl.BlockSpec(
        (LATENT, RBLK),
        lambda i, s=s: (0, jnp.minimum(s * NBLK + i, (NUM_ROWS - 1) // RBLK)))
              for s in range(4)] * 2,
    out_specs=[pl.BlockSpec((RBLK, 128), lambda i: (i, 0))] * 2,
    out_shape=[jax.ShapeDtypeStruct((VP, 128), jnp.float32)] * 2,
)


def _build_gather():
    info = plsc.get_sparse_core_info()
    nc, ns = info.num_cores, info.num_subcores
    nw = nc * ns
    b_per_w = BATCH // nw
    n_chunks = b_per_w // 128
    mesh = plsc.VectorSubcoreMesh(core_axis_name="c", subcore_axis_name="s")

    @functools.partial(
        pl.kernel,
        mesh=mesh,
        compiler_params=pltpu.CompilerParams(use_tc_tiling_on_sc=True,
                                             needs_layout_passes=False),
        out_type=[
            jax.ShapeDtypeStruct((LATENT, BATCH), jnp.float32),
            jax.ShapeDtypeStruct((LATENT, BATCH), jnp.float32),
        ],
        scratch_types=[
            pltpu.VMEM((b_per_w,), jnp.int32),
            pltpu.VMEM((n_chunks, 128), jnp.int32),
            pltpu.VMEM((b_per_w, 128), jnp.float32),
            pltpu.VMEM((LATENT, b_per_w), jnp.float32),
            pltpu.SemaphoreType.DMA,
        ],
    )
    def gather(user_hbm, item_hbm, upk_hbm, ipk_hbm, ue_out, ie_out,
               idx_v, r_v, rows_v, stage_v, sem):
        wid = lax.axis_index("s") * nc + lax.axis_index("c")
        base = pl.multiple_of(wid * b_per_w, b_per_w)
        lanes = lax.iota(jnp.int32, 16)

        def one_table(idx_hbm, pk_hbm, out_hbm):
            pltpu.sync_copy(idx_hbm.at[pl.ds(base, b_per_w)], idx_v)
            for c in range(b_per_w // 16):
                rv = idx_v[pl.ds(c * 16, 16)] & (VP - 1)
                r_v[c // 8, pl.ds((c % 8) * 16, 16)] = rv
            copies = [
                pltpu.async_copy(pk_hbm.at[r_v.at[j]],
                                 rows_v.at[pl.ds(j * 128, 128)], sem)
                for j in range(n_chunks)
            ]
            for cp in copies:
                cp.wait()

            def body(jg, carry):
                j_vec = jg * 16 + lanes
                s_vec = lax.shift_right_logical(idx_v[pl.ds(jg * 16, 16)],
                                                SBITS)
                c_base = s_vec * LATENT
                for f in range(LATENT):
                    vals = plsc.load_gather(rows_v, [j_vec, c_base + f])
                    stage_v[f, pl.ds(jg * 16, 16)] = vals
                return carry

            lax.fori_loop(0, b_per_w // 16, body, None)
            pltpu.sync_copy(stage_v, out_hbm.at[:, pl.ds(base, b_per_w)])

        one_table(user_hbm, upk_hbm, ue_out)
        one_table(item_hbm, ipk_hbm, ie_out)

    return gather


_GATHER = _build_gather()


def _mlp_body(uet_ref, iet_ref, w1a_ref, w1b_ref, b1_ref, gamma_ref, beta_ref,
              w2_ref, b2_ref, out_ref):
    dnums = (((0,), (0,)), ((), ()))
    h = (lax.dot_general(uet_ref[...], w1a_ref[...], dnums,
                         preferred_element_type=jnp.float32)
         + lax.dot_general(iet_ref[...], w1b_ref[...], dnums,
                           preferred_element_type=jnp.float32)
         + b1_ref[...])
    mean = jnp.mean(h, axis=0, keepdims=True)
    var = jnp.mean((h - mean) ** 2, axis=0, keepdims=True)
    hn = (h - mean) * lax.rsqrt(var + 1e-5) * gamma_ref[...] + beta_ref[...]
    hr = jnp.maximum(hn, 0.0)
    logit = jnp.dot(hr, w2_ref[...], preferred_element_type=jnp.float32) + b2_ref[...]
    out_ref[...] = jax.nn.sigmoid(logit)


_MLP = pl.pallas_call(
    _mlp_body,
    out_shape=jax.ShapeDtypeStruct((BATCH, 1), jnp.float32),
)


def kernel(user, item, user_table, item_table, W1, b1, gamma, beta, W2, b2):
    ut, it = user_table.T, item_table.T
    upk, ipk = _REPACK(ut, ut, ut, ut, it, it, it, it)
    uet, iet = _GATHER(user.astype(jnp.int32), item.astype(jnp.int32), upk, ipk)
    out = _MLP(
        uet, iet,
        W1[:LATENT], W1[LATENT:],
        b1.reshape(1, HIDDEN), gamma.reshape(1, HIDDEN), beta.reshape(1, HIDDEN),
        W2, b2.reshape(1, 1),
    )
    return out.reshape(BATCH)


# speedup vs baseline: 2.3654x; 1.4909x over previous
import functools

import jax
import jax.numpy as jnp
from jax import lax
from jax.experimental import pallas as pl
from jax.experimental.pallas import tpu as pltpu
from jax.experimental.pallas import tpu_sc as plsc

BATCH = 16384
LATENT = 32
HIDDEN = 128
NUM_ROWS = 1000000
VP = 262144
SBITS = 18
RBLK = 2048
NBLK = VP // RBLK


def _repack_body(u0, u1, u2, u3, i0, i1, i2, i3, uout, iout):
    frow = lax.broadcasted_iota(jnp.int32, (LATENT, 128), 0)
    jcol = lax.broadcasted_iota(jnp.int32, (LATENT, 128), 1)
    dnums = (((0,), (0,)), ((), ()))

    def pack(refs):
        acc = None
        for s, r in enumerate(refs):
            es = (jcol == s * LATENT + frow).astype(jnp.float32)
            t = lax.dot_general(r[...], es, dnums,
                                preferred_element_type=jnp.float32)
            acc = t if acc is None else acc + t
        return acc

    uout[...] = pack((u0, u1, u2, u3))
    iout[...] = pack((i0, i1, i2, i3))


_REPACK = pl.pallas_call(
    _repack_body,
    grid=(NBLK,),
    in_specs=[pl.BlockSpec(
        (LATENT, RBLK),
        lambda i, s=s: (0, jnp.minimum(s * NBLK + i, (NUM_ROWS - 1) // RBLK)))
              for s in range(4)] * 2,
    out_specs=[pl.BlockSpec((RBLK, 128), lambda i: (i, 0))] * 2,
    out_shape=[jax.ShapeDtypeStruct((VP, 128), jnp.float32)] * 2,
)


def _build_gather():
    info = plsc.get_sparse_core_info()
    nc, ns = info.num_cores, info.num_subcores
    nw = nc * ns
    b_per_w = BATCH // nw
    n_chunks = b_per_w // 128
    mesh = plsc.VectorSubcoreMesh(core_axis_name="c", subcore_axis_name="s")

    @functools.partial(
        pl.kernel,
        mesh=mesh,
        compiler_params=pltpu.CompilerParams(use_tc_tiling_on_sc=True,
                                             needs_layout_passes=False),
        out_type=[
            jax.ShapeDtypeStruct((LATENT, BATCH), jnp.float32),
            jax.ShapeDtypeStruct((LATENT, BATCH), jnp.float32),
        ],
        scratch_types=[
            pltpu.VMEM((b_per_w,), jnp.int32),
            pltpu.VMEM((n_chunks, 128), jnp.int32),
            pltpu.VMEM((b_per_w, 128), jnp.float32),
            pltpu.VMEM((LATENT, b_per_w), jnp.float32),
            pltpu.SemaphoreType.DMA,
        ],
    )
    def gather(user_hbm, item_hbm, upk_hbm, ipk_hbm, ue_out, ie_out,
               idx_v, r_v, rows_v, stage_v, sem):
        wid = lax.axis_index("s") * nc + lax.axis_index("c")
        base = pl.multiple_of(wid * b_per_w, b_per_w)
        lanes = lax.iota(jnp.int32, 16)

        def one_table(idx_hbm, pk_hbm, out_hbm):
            pltpu.sync_copy(idx_hbm.at[pl.ds(base, b_per_w)], idx_v)
            for c in range(b_per_w // 16):
                rv = idx_v[pl.ds(c * 16, 16)] & (VP - 1)
                r_v[c // 8, pl.ds((c % 8) * 16, 16)] = rv
            copies = [
                pltpu.async_copy(pk_hbm.at[r_v.at[j]],
                                 rows_v.at[pl.ds(j * 128, 128)], sem)
                for j in range(n_chunks)
            ]
            for cp in copies:
                cp.wait()

            def body(jg, carry):
                j_vec = jg * 16 + lanes
                s_vec = lax.shift_right_logical(idx_v[pl.ds(jg * 16, 16)],
                                                SBITS)
                c_base = s_vec * LATENT
                for f in range(LATENT):
                    vals = plsc.load_gather(rows_v, [j_vec, c_base + f])
                    stage_v[f, pl.ds(jg * 16, 16)] = vals
                return carry

            lax.fori_loop(0, b_per_w // 16, body, None)
            pltpu.sync_copy(stage_v, out_hbm.at[:, pl.ds(base, b_per_w)])

        one_table(user_hbm, upk_hbm, ue_out)
        one_table(item_hbm, ipk_hbm, ie_out)

    return gather


_GATHER = _build_gather()


def _mlp_body(uet_ref, iet_ref, w1a_ref, w1b_ref, b1_ref, gamma_ref, beta_ref,
              w2_ref, b2_ref, out_ref):
    dnums = (((0,), (0,)), ((), ()))
    h = (lax.dot_general(uet_ref[...], w1a_ref[...], dnums,
                         preferred_element_type=jnp.float32)
         + lax.dot_general(iet_ref[...], w1b_ref[...], dnums,
                           preferred_element_type=jnp.float32)
         + b1_ref[...])
    mean = jnp.mean(h, axis=0, keepdims=True)
    var = jnp.mean((h - mean) ** 2, axis=0, keepdims=True)
    hn = (h - mean) * lax.rsqrt(var + 1e-5) * gamma_ref[...] + beta_ref[...]
    hr = jnp.maximum(hn, 0.0)
    logit = jnp.dot(hr, w2_ref[...], preferred_element_type=jnp.float32) + b2_ref[...]
    out_ref[...] = jax.nn.sigmoid(logit)


_MLP = pl.pallas_call(
    _mlp_body,
    out_shape=jax.ShapeDtypeStruct((BATCH, 1), jnp.float32),
)


def kernel(user, item, user_table, item_table, W1, b1, gamma, beta, W2, b2):
    ut, it = user_table.T, item_table.T
    upk, ipk = _REPACK(ut, ut, ut, ut, it, it, it, it)
    uet, iet = _GATHER(user.astype(jnp.int32), item.astype(jnp.int32), upk, ipk)
    out = _MLP(
        uet, iet,
        W1[:LATENT], W1[LATENT:],
        b1.reshape(1, HIDDEN), gamma.reshape(1, HIDDEN), beta.reshape(1, HIDDEN),
        W2, b2.reshape(1, 1),
    )
    return out.reshape(BATCH)


# speedup vs baseline: 2.6056x; 1.1016x over previous
import functools

import jax
import jax.numpy as jnp
from jax import lax
from jax.experimental import pallas as pl
from jax.experimental.pallas import tpu as pltpu
from jax.experimental.pallas import tpu_sc as plsc

BATCH = 16384
LATENT = 32
HIDDEN = 128
NUM_ROWS = 1000000
VP = 131072
SBITS = 17
NS_GROUP = 8
RBLK = 4096
NBLK = VP // RBLK
MAXBLK = (NUM_ROWS - 1) // RBLK


def _repack_body(*refs):
    urefs, irefs = refs[:NS_GROUP], refs[NS_GROUP:2 * NS_GROUP]
    uout, iout = refs[2 * NS_GROUP], refs[2 * NS_GROUP + 1]
    frow = lax.broadcasted_iota(jnp.int32, (LATENT, 128), 0)
    jcol = lax.broadcasted_iota(jnp.int32, (LATENT, 128), 1)
    dnums = (((0,), (0,)), ((), ()))

    def pack(srefs):
        lo = None
        hi = None
        for s, r in enumerate(srefs):
            es_lo = ((jcol == s * 16 + frow) & (frow < 16)).astype(jnp.float32)
            es_hi = ((jcol == s * 16 + frow - 16) & (frow >= 16)).astype(jnp.float32)
            x = r[...]
            tl = lax.dot_general(x, es_lo, dnums,
                                 preferred_element_type=jnp.float32)
            th = lax.dot_general(x, es_hi, dnums,
                                 preferred_element_type=jnp.float32)
            lo = tl if lo is None else lo + tl
            hi = th if hi is None else hi + th
        lo_b = lax.bitcast_convert_type(lo, jnp.uint32)
        hi_b = lax.bitcast_convert_type(hi, jnp.uint32)
        pair = (hi_b & jnp.uint32(0xFFFF0000)) | (lo_b >> 16)
        return lax.bitcast_convert_type(pair, jnp.float32)

    uout[...] = pack(urefs)
    iout[...] = pack(irefs)


_REPACK = pl.pallas_call(
    _repack_body,
    grid=(NBLK,),
    in_specs=[pl.BlockSpec(
        (LATENT, RBLK),
        lambda i, s=s: (0, jnp.minimum(s * NBLK + i, MAXBLK)))
        for s in range(NS_GROUP)] * 2,
    out_specs=[pl.BlockSpec((RBLK, 128), lambda i: (i, 0))] * 2,
    out_shape=[jax.ShapeDtypeStruct((VP, 128), jnp.float32)] * 2,
)


def _build_gather():
    info = plsc.get_sparse_core_info()
    nc, ns = info.num_cores, info.num_subcores
    nw = nc * ns
    b_per_w = BATCH // nw
    n_chunks = b_per_w // 128
    mesh = plsc.VectorSubcoreMesh(core_axis_name="c", subcore_axis_name="s")

    @functools.partial(
        pl.kernel,
        mesh=mesh,
        compiler_params=pltpu.CompilerParams(use_tc_tiling_on_sc=True,
                                             needs_layout_passes=False),
        out_type=[
            jax.ShapeDtypeStruct((LATENT, BATCH), jnp.float32),
            jax.ShapeDtypeStruct((LATENT, BATCH), jnp.float32),
        ],
        scratch_types=[
            pltpu.VMEM((b_per_w,), jnp.int32),
            pltpu.VMEM((n_chunks, 128), jnp.int32),
            pltpu.VMEM((b_per_w, 128), jnp.float32),
            pltpu.VMEM((LATENT, b_per_w), jnp.float32),
            pltpu.SemaphoreType.DMA,
        ],
    )
    def gather(user_hbm, item_hbm, upk_hbm, ipk_hbm, ue_out, ie_out,
               idx_v, r_v, rows_v, stage_v, sem):
        wid = lax.axis_index("s") * nc + lax.axis_index("c")
        base = pl.multiple_of(wid * b_per_w, b_per_w)
        lanes = lax.iota(jnp.int32, 16)

        def one_table(idx_hbm, pk_hbm, out_hbm):
            pltpu.sync_copy(idx_hbm.at[pl.ds(base, b_per_w)], idx_v)
            for c in range(b_per_w // 16):
                rv = idx_v[pl.ds(c * 16, 16)] & (VP - 1)
                r_v[c // 8, pl.ds((c % 8) * 16, 16)] = rv
            copies = [
                pltpu.async_copy(pk_hbm.at[r_v.at[j]],
                                 rows_v.at[pl.ds(j * 128, 128)], sem)
                for j in range(n_chunks)
            ]
            for cp in copies:
                cp.wait()

            def body(jg, carry):
                j_vec = jg * 16 + lanes
                s_vec = lax.shift_right_logical(idx_v[pl.ds(jg * 16, 16)],
                                                SBITS)
                c_base = s_vec * 16
                for f2 in range(16):
                    vals = plsc.load_gather(rows_v, [j_vec, c_base + f2])
                    bits = lax.bitcast_convert_type(vals, jnp.uint32)
                    lo = lax.bitcast_convert_type(bits << 16, jnp.float32)
                    hi = lax.bitcast_convert_type(
                        bits & jnp.uint32(0xFFFF0000), jnp.float32)
                    stage_v[f2, pl.ds(jg * 16, 16)] = lo
                    stage_v[f2 + 16, pl.ds(jg * 16, 16)] = hi
                return carry

            lax.fori_loop(0, b_per_w // 16, body, None)
            pltpu.sync_copy(stage_v, out_hbm.at[:, pl.ds(base, b_per_w)])

        one_table(user_hbm, upk_hbm, ue_out)
        one_table(item_hbm, ipk_hbm, ie_out)

    return gather


_GATHER = _build_gather()


def _mlp_body(uet_ref, iet_ref, w1a_ref, w1b_ref, b1_ref, gamma_ref, beta_ref,
              w2_ref, b2_ref, out_ref):
    dnums = (((0,), (0,)), ((), ()))
    h = (lax.dot_general(uet_ref[...], w1a_ref[...], dnums,
                         preferred_element_type=jnp.float32)
         + lax.dot_general(iet_ref[...], w1b_ref[...], dnums,
                           preferred_element_type=jnp.float32)
         + b1_ref[...])
    mean = jnp.mean(h, axis=0, keepdims=True)
    var = jnp.mean((h - mean) ** 2, axis=0, keepdims=True)
    hn = (h - mean) * lax.rsqrt(var + 1e-5) * gamma_ref[...] + beta_ref[...]
    hr = jnp.maximum(hn, 0.0)
    logit = jnp.dot(hr, w2_ref[...], preferred_element_type=jnp.float32) + b2_ref[...]
    out_ref[...] = jax.nn.sigmoid(logit)


_MLP = pl.pallas_call(
    _mlp_body,
    out_shape=jax.ShapeDtypeStruct((BATCH, 1), jnp.float32),
)


def kernel(user, item, user_table, item_table, W1, b1, gamma, beta, W2, b2):
    ut, it = user_table.T, item_table.T
    upk, ipk = _REPACK(*([ut] * NS_GROUP), *([it] * NS_GROUP))
    uet, iet = _GATHER(user.astype(jnp.int32), item.astype(jnp.int32), upk, ipk)
    out = _MLP(
        uet, iet,
        W1[:LATENT], W1[LATENT:],
        b1.reshape(1, HIDDEN), gamma.reshape(1, HIDDEN), beta.reshape(1, HIDDEN),
        W2, b2.reshape(1, 1),
    )
    return out.reshape(BATCH)


# speedup vs baseline: 5.0551x; 1.9401x over previous
import functools

import jax
import jax.numpy as jnp
from jax import lax
from jax.experimental import pallas as pl
from jax.experimental.pallas import tpu as pltpu
from jax.experimental.pallas import tpu_sc as plsc

BATCH = 16384
LATENT = 32
HIDDEN = 128
NUM_ROWS = 1000000
VP = 131072
SBITS = 17
NS_GROUP = 8
RBLK = 4096
NBLK = VP // RBLK
MAXBLK = (NUM_ROWS - 1) // RBLK


def _repack_body(*refs):
    urefs, irefs = refs[:NS_GROUP], refs[NS_GROUP:2 * NS_GROUP]
    uout, iout = refs[2 * NS_GROUP], refs[2 * NS_GROUP + 1]
    k = NS_GROUP * LATENT
    frow = lax.broadcasted_iota(jnp.int32, (k, 128), 0)
    jcol = lax.broadcasted_iota(jnp.int32, (k, 128), 1)
    s = frow // LATENT
    f = frow % LATENT
    e_lo = ((f < 16) & (jcol == s * 16 + f)).astype(jnp.bfloat16)
    e_hi = ((f >= 16) & (jcol == s * 16 + f - 16)).astype(jnp.bfloat16)
    dnums = (((0,), (0,)), ((), ()))

    def pack(srefs):
        x = jnp.concatenate([r[...] for r in srefs],
                            axis=0).astype(jnp.bfloat16)
        lo = lax.dot_general(x, e_lo, dnums,
                             preferred_element_type=jnp.float32)
        hi = lax.dot_general(x, e_hi, dnums,
                             preferred_element_type=jnp.float32)
        lo_b = lax.bitcast_convert_type(lo, jnp.uint32)
        hi_b = lax.bitcast_convert_type(hi, jnp.uint32)
        pair = (hi_b & jnp.uint32(0xFFFF0000)) | (lo_b >> 16)
        return lax.bitcast_convert_type(pair, jnp.float32)

    uout[...] = pack(urefs)
    iout[...] = pack(irefs)


_REPACK = pl.pallas_call(
    _repack_body,
    grid=(NBLK,),
    in_specs=[pl.BlockSpec(
        (LATENT, RBLK),
        lambda i, s=s: (0, jnp.minimum(s * NBLK + i, MAXBLK)))
        for s in range(NS_GROUP)] * 2,
    out_specs=[pl.BlockSpec((RBLK, 128), lambda i: (i, 0))] * 2,
    out_shape=[jax.ShapeDtypeStruct((VP, 128), jnp.float32)] * 2,
)


def _build_gather():
    info = plsc.get_sparse_core_info()
    nc, ns = info.num_cores, info.num_subcores
    nw = nc * ns
    b_per_w = BATCH // nw
    n_chunks = b_per_w // 128
    mesh = plsc.VectorSubcoreMesh(core_axis_name="c", subcore_axis_name="s")

    @functools.partial(
        pl.kernel,
        mesh=mesh,
        compiler_params=pltpu.CompilerParams(use_tc_tiling_on_sc=True,
                                             needs_layout_passes=False),
        out_type=[
            jax.ShapeDtypeStruct((LATENT, BATCH), jnp.float32),
            jax.ShapeDtypeStruct((LATENT, BATCH), jnp.float32),
        ],
        scratch_types=[
            pltpu.VMEM((b_per_w,), jnp.int32),
            pltpu.VMEM((n_chunks, 128), jnp.int32),
            pltpu.VMEM((b_per_w, 128), jnp.float32),
            pltpu.VMEM((LATENT, b_per_w), jnp.float32),
            pltpu.SemaphoreType.DMA,
        ],
    )
    def gather(user_hbm, item_hbm, upk_hbm, ipk_hbm, ue_out, ie_out,
               idx_v, r_v, rows_v, stage_v, sem):
        wid = lax.axis_index("s") * nc + lax.axis_index("c")
        base = pl.multiple_of(wid * b_per_w, b_per_w)
        lanes = lax.iota(jnp.int32, 16)

        def one_table(idx_hbm, pk_hbm, out_hbm):
            pltpu.sync_copy(idx_hbm.at[pl.ds(base, b_per_w)], idx_v)
            for c in range(b_per_w // 16):
                rv = idx_v[pl.ds(c * 16, 16)] & (VP - 1)
                r_v[c // 8, pl.ds((c % 8) * 16, 16)] = rv
            copies = [
                pltpu.async_copy(pk_hbm.at[r_v.at[j]],
                                 rows_v.at[pl.ds(j * 128, 128)], sem)
                for j in range(n_chunks)
            ]
            for cp in copies:
                cp.wait()

            def body(jg, carry):
                j_vec = jg * 16 + lanes
                s_vec = lax.shift_right_logical(idx_v[pl.ds(jg * 16, 16)],
                                                SBITS)
                c_base = s_vec * 16
                for f2 in range(16):
                    vals = plsc.load_gather(rows_v, [j_vec, c_base + f2])
                    bits = lax.bitcast_convert_type(vals, jnp.uint32)
                    lo = lax.bitcast_convert_type(bits << 16, jnp.float32)
                    hi = lax.bitcast_convert_type(
                        bits & jnp.uint32(0xFFFF0000), jnp.float32)
                    stage_v[f2, pl.ds(jg * 16, 16)] = lo
                    stage_v[f2 + 16, pl.ds(jg * 16, 16)] = hi
                return carry

            lax.fori_loop(0, b_per_w // 16, body, None)
            pltpu.sync_copy(stage_v, out_hbm.at[:, pl.ds(base, b_per_w)])

        one_table(user_hbm, upk_hbm, ue_out)
        one_table(item_hbm, ipk_hbm, ie_out)

    return gather


_GATHER = _build_gather()


def _mlp_body(uet_ref, iet_ref, w1a_ref, w1b_ref, b1_ref, gamma_ref, beta_ref,
              w2_ref, b2_ref, out_ref):
    dnums = (((0,), (0,)), ((), ()))
    h = (lax.dot_general(uet_ref[...], w1a_ref[...], dnums,
                         preferred_element_type=jnp.float32)
         + lax.dot_general(iet_ref[...], w1b_ref[...], dnums,
                           preferred_element_type=jnp.float32)
         + b1_ref[...])
    mean = jnp.mean(h, axis=0, keepdims=True)
    var = jnp.mean((h - mean) ** 2, axis=0, keepdims=True)
    hn = (h - mean) * lax.rsqrt(var + 1e-5) * gamma_ref[...] + beta_ref[...]
    hr = jnp.maximum(hn, 0.0)
    logit = jnp.dot(hr, w2_ref[...], preferred_element_type=jnp.float32) + b2_ref[...]
    out_ref[...] = jax.nn.sigmoid(logit)


_MLP = pl.pallas_call(
    _mlp_body,
    out_shape=jax.ShapeDtypeStruct((BATCH, 1), jnp.float32),
)


def kernel(user, item, user_table, item_table, W1, b1, gamma, beta, W2, b2):
    ut, it = user_table.T, item_table.T
    upk, ipk = _REPACK(*([ut] * NS_GROUP), *([it] * NS_GROUP))
    uet, iet = _GATHER(user.astype(jnp.int32), item.astype(jnp.int32), upk, ipk)
    out = _MLP(
        uet, iet,
        W1[:LATENT], W1[LATENT:],
        b1.reshape(1, HIDDEN), gamma.reshape(1, HIDDEN), beta.reshape(1, HIDDEN),
        W2, b2.reshape(1, 1),
    )
    return out.reshape(BATCH)


# speedup vs baseline: 5.3061x; 1.0496x over previous
import functools

import jax
import jax.numpy as jnp
from jax import lax
from jax.experimental import pallas as pl
from jax.experimental.pallas import tpu as pltpu
from jax.experimental.pallas import tpu_sc as plsc

BATCH = 16384
LATENT = 32
HIDDEN = 128
NUM_ROWS = 1000000
VP = 131072
SBITS = 17
NS_GROUP = 8
RBLK = 4096
NBLK = VP // RBLK
MAXBLK = (NUM_ROWS - 1) // RBLK


def _repack_body(*refs):
    urefs, irefs = refs[:NS_GROUP], refs[NS_GROUP:2 * NS_GROUP]
    uout, iout = refs[2 * NS_GROUP], refs[2 * NS_GROUP + 1]
    k = NS_GROUP * LATENT
    frow = lax.broadcasted_iota(jnp.int32, (k, 128), 0)
    jcol = lax.broadcasted_iota(jnp.int32, (k, 128), 1)
    s = frow // LATENT
    f = frow % LATENT
    e_lo = ((f < 16) & (jcol == s * 16 + f)).astype(jnp.bfloat16)
    e_hi = ((f >= 16) & (jcol == s * 16 + f - 16)).astype(jnp.bfloat16)
    dnums = (((0,), (0,)), ((), ()))

    def pack(srefs):
        x = jnp.concatenate([r[...] for r in srefs],
                            axis=0).astype(jnp.bfloat16)
        lo = lax.dot_general(x, e_lo, dnums,
                             preferred_element_type=jnp.float32)
        hi = lax.dot_general(x, e_hi, dnums,
                             preferred_element_type=jnp.float32)
        lo_b = lax.bitcast_convert_type(lo, jnp.uint32)
        hi_b = lax.bitcast_convert_type(hi, jnp.uint32)
        pair = (hi_b & jnp.uint32(0xFFFF0000)) | (lo_b >> 16)
        return lax.bitcast_convert_type(pair, jnp.float32)

    uout[...] = pack(urefs)
    iout[...] = pack(irefs)


_REPACK = pl.pallas_call(
    _repack_body,
    grid=(NBLK,),
    in_specs=[pl.BlockSpec(
        (LATENT, RBLK),
        lambda i, s=s: (0, jnp.minimum(s * NBLK + i, MAXBLK)))
        for s in range(NS_GROUP)] * 2,
    out_specs=[pl.BlockSpec((RBLK, 128), lambda i: (i, 0))] * 2,
    out_shape=[jax.ShapeDtypeStruct((VP, 128), jnp.float32)] * 2,
)


def _build_gather():
    info = plsc.get_sparse_core_info()
    nc, ns = info.num_cores, info.num_subcores
    nw = nc * ns
    b_per_w = BATCH // nw
    n_chunks = b_per_w // 128
    mesh = plsc.VectorSubcoreMesh(core_axis_name="c", subcore_axis_name="s")

    @functools.partial(
        pl.kernel,
        mesh=mesh,
        compiler_params=pltpu.CompilerParams(use_tc_tiling_on_sc=True,
                                             needs_layout_passes=False),
        out_type=[
            jax.ShapeDtypeStruct((LATENT, BATCH), jnp.float32),
            jax.ShapeDtypeStruct((LATENT, BATCH), jnp.float32),
        ],
        scratch_types=[
            pltpu.VMEM((b_per_w,), jnp.int32),
            pltpu.VMEM((n_chunks, 128), jnp.int32),
            pltpu.VMEM((b_per_w, 128), jnp.float32),
            pltpu.VMEM((LATENT, b_per_w), jnp.float32),
            pltpu.SemaphoreType.DMA,
        ],
    )
    def gather(user_hbm, item_hbm, upk_hbm, ipk_hbm, ue_out, ie_out,
               idx_v, r_v, rows_v, stage_v, sem):
        wid = lax.axis_index("s") * nc + lax.axis_index("c")
        base = pl.multiple_of(wid * b_per_w, b_per_w)
        lanes = lax.iota(jnp.int32, 16)

        def one_table(idx_hbm, pk_hbm, out_hbm):
            pltpu.sync_copy(idx_hbm.at[pl.ds(base, b_per_w)], idx_v)
            for c in range(b_per_w // 16):
                rv = idx_v[pl.ds(c * 16, 16)] & (VP - 1)
                r_v[c // 8, pl.ds((c % 8) * 16, 16)] = rv
            copies = [
                pltpu.async_copy(pk_hbm.at[r_v.at[j]],
                                 rows_v.at[pl.ds(j * 128, 128)], sem)
                for j in range(n_chunks)
            ]
            for cp in copies:
                cp.wait()

            def body(jg, carry):
                j_vec = jg * 16 + lanes
                s_vec = lax.shift_right_logical(idx_v[pl.ds(jg * 16, 16)],
                                                SBITS)
                c_base = s_vec * 16
                for f2 in range(16):
                    vals = plsc.load_gather(rows_v, [j_vec, c_base + f2])
                    bits = lax.bitcast_convert_type(vals, jnp.uint32)
                    lo = lax.bitcast_convert_type(bits << 16, jnp.float32)
                    hi = lax.bitcast_convert_type(
                        bits & jnp.uint32(0xFFFF0000), jnp.float32)
                    stage_v[f2, pl.ds(jg * 16, 16)] = lo
                    stage_v[f2 + 16, pl.ds(jg * 16, 16)] = hi
                return carry

            lax.fori_loop(0, b_per_w // 16, body, None)
            pltpu.sync_copy(stage_v, out_hbm.at[:, pl.ds(base, b_per_w)])

        one_table(user_hbm, upk_hbm, ue_out)
        one_table(item_hbm, ipk_hbm, ie_out)

    return gather


_GATHER = _build_gather()


def _mlp_body(uet_ref, iet_ref, w1a_ref, w1b_ref, b1_ref, gamma_ref, beta_ref,
              w2_ref, b2_ref, out_ref):
    dnums = (((0,), (0,)), ((), ()))
    h = (lax.dot_general(uet_ref[...].astype(jnp.bfloat16),
                         w1a_ref[...].astype(jnp.bfloat16), dnums,
                         preferred_element_type=jnp.float32)
         + lax.dot_general(iet_ref[...].astype(jnp.bfloat16),
                           w1b_ref[...].astype(jnp.bfloat16), dnums,
                           preferred_element_type=jnp.float32)
         + b1_ref[...])
    mean = jnp.mean(h, axis=0, keepdims=True)
    var = jnp.mean((h - mean) ** 2, axis=0, keepdims=True)
    hn = (h - mean) * lax.rsqrt(var + 1e-5) * gamma_ref[...] + beta_ref[...]
    hr = jnp.maximum(hn, 0.0)
    logit = lax.dot_general(
        w2_ref[...], hr, (((0,), (1,)), ((), ())),
        preferred_element_type=jnp.float32) + b2_ref[...]
    out_ref[...] = jax.nn.sigmoid(logit)


_MLP = pl.pallas_call(
    _mlp_body,
    out_shape=jax.ShapeDtypeStruct((1, BATCH), jnp.float32),
)


def kernel(user, item, user_table, item_table, W1, b1, gamma, beta, W2, b2):
    ut, it = user_table.T, item_table.T
    upk, ipk = _REPACK(*([ut] * NS_GROUP), *([it] * NS_GROUP))
    uet, iet = _GATHER(user.astype(jnp.int32), item.astype(jnp.int32), upk, ipk)
    out = _MLP(
        uet, iet,
        W1[:LATENT], W1[LATENT:],
        b1.reshape(1, HIDDEN), gamma.reshape(1, HIDDEN), beta.reshape(1, HIDDEN),
        W2, b2.reshape(1, 1),
    )
    return out.reshape(BATCH)
